# baseline (speedup 1.0000x reference)
import functools
import jax
import jax.numpy as jnp
from jax import lax
from jax.experimental import pallas as pl
from jax.experimental.pallas import tpu as pltpu
from jax.experimental.pallas import tpu_sc as plsc

_B = 4
_D = 128
_T = 1024
_P = 256 * 256

_NC_CORES = 2
_NSUB = 16
_NW = _NC_CORES * _NSUB
_PPW = _P // _NW
_DB = 16
_NDB = _D // _DB
_C = 512
_NCH = _PPW // _C
_L = 16


def _dslab(b):
    return _DB * _T


def _body(t01_hbm, t23_hbm, k0_hbm, alpha_hbm, valid_hbm, apod_hbm, out_hbm,
          tp01, tp23, k0cA, acA, vcA, k0cB, acB, vcB,
          acc, apod_v, semA, semB):
    tabs = (tp01, tp23)
    tab_hbms = (t01_hbm, t23_hbm)
    bufsA = (k0cA, acA, vcA)
    bufsB = (k0cB, acB, vcB)
    cid = lax.axis_index("c")
    sid = lax.axis_index("s")
    wid = sid * _NC_CORES + cid
    pbase = wid * _PPW

    lane = lax.iota(jnp.int32, _L)
    dlane_off = lane * _T
    masks = [lane == j for j in range(_L)]
    zero16 = jnp.zeros((_L,), jnp.float32)
    _HIMASK = jnp.int32(-65536)

    pltpu.sync_copy(apod_hbm, apod_v)
    tv = zero16
    for j in range(_D // _L):
        tv = tv + apod_v[pl.ds(j * _L, _L)]
    tot = jnp.full((_L,), jnp.sum(tv), jnp.float32)
    inv = jnp.ones((_L,), jnp.float32) / jnp.maximum(
        tot, jnp.finfo(jnp.float32).tiny)

    def _zero(i, carry):
        acc[pl.ds(i * _L, _L)] = zero16
        return carry
    lax.fori_loop(0, (_B * _PPW) // _L, _zero, 0)

    def _slices(gi):
        d0 = (gi // _NCH) * _DB
        p0 = pbase + (gi % _NCH) * _C
        return d0, p0

    def _issue_lut(gi, bufs, sem):
        d0, p0 = _slices(gi)
        pltpu.async_copy(k0_hbm.at[pl.ds(p0, _C), pl.ds(d0, _DB)],
                         bufs[0], sem)
        pltpu.async_copy(alpha_hbm.at[pl.ds(p0, _C), pl.ds(d0, _DB)],
                         bufs[1], sem)
        pltpu.async_copy(valid_hbm.at[pl.ds(p0, _C), pl.ds(d0, _DB)],
                         bufs[2], sem)

    def _wait_lut(gi, bufs, sem):
        d0, p0 = _slices(gi)
        pltpu.make_async_copy(k0_hbm.at[pl.ds(p0, _C), pl.ds(d0, _DB)],
                              bufs[0], sem).wait()
        pltpu.make_async_copy(alpha_hbm.at[pl.ds(p0, _C), pl.ds(d0, _DB)],
                              bufs[1], sem).wait()
        pltpu.make_async_copy(valid_hbm.at[pl.ds(p0, _C), pl.ds(d0, _DB)],
                              bufs[2], sem).wait()

    def _load_tables(gi):
        d0, _ = _slices(gi)
        for q in range(2):
            pltpu.sync_copy(tab_hbms[q].at[pl.ds(d0 * _T, _DB * _T)], tabs[q])

    def _compute(gi, bufs):
        k0c, ac, vc = bufs
        d0, _ = _slices(gi)
        ci = gi % _NCH
        wv = apod_v[pl.ds(d0, _DB)] * inv

        @plsc.parallel_loop(0, _C // _L, unroll=4)
        def _group_body(g):
            vecs = [zero16, zero16, zero16, zero16]
            for i2 in range(_L):
                i = g * _L + i2
                kv = k0c[i]
                av = ac[i]
                vv = vc[i]
                idx0 = kv + dlane_off
                idx1 = idx0 + 1
                w = vv * wv
                c1 = w * av
                c0 = w - c1
                for q in range(2):
                    w0 = plsc.load_gather(tabs[q], [idx0])
                    w1 = plsc.load_gather(tabs[q], [idx1])
                    s0_lo = plsc.bitcast(w0 << 16, jnp.float32)
                    s0_hi = plsc.bitcast(w0 & _HIMASK, jnp.float32)
                    s1_lo = plsc.bitcast(w1 << 16, jnp.float32)
                    s1_hi = plsc.bitcast(w1 & _HIMASK, jnp.float32)
                    s_lo = jnp.sum(c0 * s0_lo + c1 * s1_lo)
                    s_hi = jnp.sum(c0 * s0_hi + c1 * s1_hi)
                    vecs[2 * q] = jnp.where(masks[i2], s_lo, vecs[2 * q])
                    vecs[2 * q + 1] = jnp.where(masks[i2], s_hi,
                                                vecs[2 * q + 1])
            for b in range(_B):
                plsc.addupdate(
                    acc.at[pl.ds(b * _PPW + ci * _C + g * _L, _L)], vecs[b])

    _NPAIR = (_NDB * _NCH) // 2

    _issue_lut(0, bufsA, semA)

    def _pair_body(j, carry):
        gi0 = 2 * j
        gi1 = gi0 + 1
        _issue_lut(gi1, bufsB, semB)
        _wait_lut(gi0, bufsA, semA)

        @pl.when(gi0 % _NCH == 0)
        def _():
            _load_tables(gi0)

        _compute(gi0, bufsA)

        @pl.when(j < _NPAIR - 1)
        def _():
            _issue_lut(gi0 + 2, bufsA, semA)

        _wait_lut(gi1, bufsB, semB)
        _compute(gi1, bufsB)
        return carry

    lax.fori_loop(0, _NPAIR, _pair_body, 0)

    for b in range(_B):
        pltpu.sync_copy(acc.at[pl.ds(b * _PPW, _PPW)],
                        out_hbm.at[b, pl.ds(pbase, _PPW)])


@jax.jit
def _das(t01, t23, k0f, alphaf, validf, apod):
    mesh = plsc.VectorSubcoreMesh(core_axis_name="c", subcore_axis_name="s",
                                  num_cores=_NC_CORES, num_subcores=_NSUB)
    f = pl.kernel(
        _body,
        out_type=jax.ShapeDtypeStruct((_B, _P), jnp.float32),
        mesh=mesh,
        compiler_params=pltpu.CompilerParams(use_tc_tiling_on_sc=False,
                                             needs_layout_passes=False),
        scratch_types=[
            pltpu.VMEM((_DB * _T,), jnp.int32),
            pltpu.VMEM((_DB * _T,), jnp.int32),
            pltpu.VMEM((_C, _DB), jnp.int32),
            pltpu.VMEM((_C, _DB), jnp.float32),
            pltpu.VMEM((_C, _DB), jnp.float32),
            pltpu.VMEM((_C, _DB), jnp.int32),
            pltpu.VMEM((_C, _DB), jnp.float32),
            pltpu.VMEM((_C, _DB), jnp.float32),
            pltpu.VMEM((_B * _PPW,), jnp.float32),
            pltpu.VMEM((_D,), jnp.float32),
            pltpu.SemaphoreType.DMA,
            pltpu.SemaphoreType.DMA,
        ],
    )
    return f(t01, t23, k0f, alphaf, validf, apod)


def kernel(sino, k0, alpha, valid, apod):
    s16 = sino.reshape(_B, _D * _T).astype(jnp.bfloat16)
    u = lax.bitcast_convert_type(s16, jnp.uint16).astype(jnp.uint32)
    t01 = lax.bitcast_convert_type(u[0] | (u[1] << 16), jnp.int32)
    t23 = lax.bitcast_convert_type(u[2] | (u[3] << 16), jnp.int32)
    k0f = k0.reshape(_P, _D)
    alphaf = alpha.reshape(_P, _D)
    validf = valid.reshape(_P, _D).astype(jnp.float32)
    out = _das(t01, t23, k0f, alphaf, validf, apod)
    return out.reshape(_B, 1, 256, 256)


# speedup vs baseline: 1.1604x; 1.1604x over previous
import functools
import jax
import jax.numpy as jnp
from jax import lax
from jax.experimental import pallas as pl
from jax.experimental.pallas import tpu as pltpu
from jax.experimental.pallas import tpu_sc as plsc

_B = 4
_D = 128
_T = 1024
_P = 256 * 256

_NC_CORES = 2
_NSUB = 16
_NW = _NC_CORES * _NSUB
_PPW = _P // _NW
_DB = 16
_NDB = _D // _DB
_C = 512
_NCH = _PPW // _C
_L = 16


def _dslab(b):
    return _DB * _T


def _body(t01_hbm, t23_hbm, k0_hbm, alpha_hbm, valid_hbm, apod_hbm, out_hbm,
          tp01, tp23, k0cA, acA, vcA, k0cB, acB, vcB,
          acc, apod_v, semA, semB):
    tabs = (tp01, tp23)
    tab_hbms = (t01_hbm, t23_hbm)
    bufsA = (k0cA, acA, vcA)
    bufsB = (k0cB, acB, vcB)
    cid = lax.axis_index("c")
    sid = lax.axis_index("s")
    wid = sid * _NC_CORES + cid
    pbase = wid * _PPW

    lane = lax.iota(jnp.int32, _L)
    dlane_off = lane * _T
    masks = [lane == j for j in range(_L)]
    zero16 = jnp.zeros((_L,), jnp.float32)
    _HIMASK = jnp.int32(-65536)

    pltpu.sync_copy(apod_hbm, apod_v)
    tv = zero16
    for j in range(_D // _L):
        tv = tv + apod_v[pl.ds(j * _L, _L)]
    tot = jnp.full((_L,), jnp.sum(tv), jnp.float32)
    inv = jnp.ones((_L,), jnp.float32) / jnp.maximum(
        tot, jnp.finfo(jnp.float32).tiny)

    def _zero(i, carry):
        acc[pl.ds(i * _L, _L)] = zero16
        return carry
    lax.fori_loop(0, (_B * _PPW) // _L, _zero, 0)

    def _slices(gi):
        d0 = (gi // _NCH) * _DB
        p0 = pbase + (gi % _NCH) * _C
        return d0, p0

    def _issue_lut(gi, bufs, sem):
        d0, p0 = _slices(gi)
        pltpu.async_copy(k0_hbm.at[pl.ds(p0, _C), pl.ds(d0, _DB)],
                         bufs[0], sem)
        pltpu.async_copy(alpha_hbm.at[pl.ds(p0, _C), pl.ds(d0, _DB)],
                         bufs[1], sem)
        pltpu.async_copy(valid_hbm.at[pl.ds(p0, _C), pl.ds(d0, _DB)],
                         bufs[2], sem)

    def _wait_lut(gi, bufs, sem):
        d0, p0 = _slices(gi)
        pltpu.make_async_copy(k0_hbm.at[pl.ds(p0, _C), pl.ds(d0, _DB)],
                              bufs[0], sem).wait()
        pltpu.make_async_copy(alpha_hbm.at[pl.ds(p0, _C), pl.ds(d0, _DB)],
                              bufs[1], sem).wait()
        pltpu.make_async_copy(valid_hbm.at[pl.ds(p0, _C), pl.ds(d0, _DB)],
                              bufs[2], sem).wait()

    def _load_tables(gi):
        d0, _ = _slices(gi)
        for q in range(2):
            pltpu.sync_copy(tab_hbms[q].at[pl.ds(d0 * _T, _DB * _T)], tabs[q])

    def _compute(gi, bufs):
        k0c, ac, vc = bufs
        d0, _ = _slices(gi)
        ci = gi % _NCH
        wv = apod_v[pl.ds(d0, _DB)] * inv

        @plsc.parallel_loop(0, _C // _L, unroll=4)
        def _group_body(g):
            vecs = [zero16, zero16, zero16, zero16]
            for i2 in range(_L):
                i = g * _L + i2
                kv = k0c[i]
                av = ac[i]
                vv = vc[i]
                idx0 = kv + dlane_off
                idx1 = idx0 + 1
                w = vv * wv
                c1 = w * av
                c0 = w - c1
                c0p = plsc.pack(c0, c0, format=plsc.PackFormat.INTERLEAVED)
                c1p = plsc.pack(c1, c1, format=plsc.PackFormat.INTERLEAVED)
                for q in range(2):
                    w0 = plsc.load_gather(tabs[q], [idx0])
                    w1 = plsc.load_gather(tabs[q], [idx1])
                    b0 = plsc.bitcast(w0, jnp.bfloat16)
                    b1 = plsc.bitcast(w1, jnp.bfloat16)
                    t = c0p * b0 + c1p * b1
                    te, to = plsc.unpack(t, format=plsc.PackFormat.INTERLEAVED)
                    s_lo = jnp.sum(te)
                    s_hi = jnp.sum(to)
                    vecs[2 * q] = jnp.where(masks[i2], s_lo, vecs[2 * q])
                    vecs[2 * q + 1] = jnp.where(masks[i2], s_hi,
                                                vecs[2 * q + 1])
            for b in range(_B):
                plsc.addupdate(
                    acc.at[pl.ds(b * _PPW + ci * _C + g * _L, _L)], vecs[b])

    _NPAIR = (_NDB * _NCH) // 2

    _issue_lut(0, bufsA, semA)

    def _pair_body(j, carry):
        gi0 = 2 * j
        gi1 = gi0 + 1
        _issue_lut(gi1, bufsB, semB)
        _wait_lut(gi0, bufsA, semA)

        @pl.when(gi0 % _NCH == 0)
        def _():
            _load_tables(gi0)

        _compute(gi0, bufsA)

        @pl.when(j < _NPAIR - 1)
        def _():
            _issue_lut(gi0 + 2, bufsA, semA)

        _wait_lut(gi1, bufsB, semB)
        _compute(gi1, bufsB)
        return carry

    lax.fori_loop(0, _NPAIR, _pair_body, 0)

    for b in range(_B):
        pltpu.sync_copy(acc.at[pl.ds(b * _PPW, _PPW)],
                        out_hbm.at[b, pl.ds(pbase, _PPW)])


@jax.jit
def _das(t01, t23, k0f, alphaf, validf, apod):
    mesh = plsc.VectorSubcoreMesh(core_axis_name="c", subcore_axis_name="s",
                                  num_cores=_NC_CORES, num_subcores=_NSUB)
    f = pl.kernel(
        _body,
        out_type=jax.ShapeDtypeStruct((_B, _P), jnp.float32),
        mesh=mesh,
        compiler_params=pltpu.CompilerParams(use_tc_tiling_on_sc=False,
                                             needs_layout_passes=False),
        scratch_types=[
            pltpu.VMEM((_DB * _T,), jnp.int32),
            pltpu.VMEM((_DB * _T,), jnp.int32),
            pltpu.VMEM((_C, _DB), jnp.int32),
            pltpu.VMEM((_C, _DB), jnp.float32),
            pltpu.VMEM((_C, _DB), jnp.float32),
            pltpu.VMEM((_C, _DB), jnp.int32),
            pltpu.VMEM((_C, _DB), jnp.float32),
            pltpu.VMEM((_C, _DB), jnp.float32),
            pltpu.VMEM((_B * _PPW,), jnp.float32),
            pltpu.VMEM((_D,), jnp.float32),
            pltpu.SemaphoreType.DMA,
            pltpu.SemaphoreType.DMA,
        ],
    )
    return f(t01, t23, k0f, alphaf, validf, apod)


def kernel(sino, k0, alpha, valid, apod):
    s16 = sino.reshape(_B, _D * _T).astype(jnp.bfloat16)
    u = lax.bitcast_convert_type(s16, jnp.uint16).astype(jnp.uint32)
    t01 = lax.bitcast_convert_type(u[0] | (u[1] << 16), jnp.int32)
    t23 = lax.bitcast_convert_type(u[2] | (u[3] << 16), jnp.int32)
    k0f = k0.reshape(_P, _D)
    alphaf = alpha.reshape(_P, _D)
    validf = valid.reshape(_P, _D).astype(jnp.float32)
    out = _das(t01, t23, k0f, alphaf, validf, apod)
    return out.reshape(_B, 1, 256, 256)
